# Initial kernel scaffold; baseline (speedup 1.0000x reference)
import jax
import jax.numpy as jnp
from jax.experimental import pallas as pl


def kernel(x, labels, Mem):
    raise NotImplementedError("write your pallas kernel here")


# speedup vs baseline: 17.1566x; 17.1566x over previous
import functools

import jax
import jax.numpy as jnp
from jax import lax
from jax.experimental import pallas as pl
from jax.experimental.pallas import tpu as pltpu
from jax.experimental.pallas import tpu_sc as plsc

B = 1024
M = 100000
FD = 512
K = 1000
MARGIN = 1.0

TM = 2048
NT = 49
MP = NT * TM

NB = 1024
Z_LO = 3.5
Z_HI = 1.25
BIG = 1.0e9

NW = 32
RPW = B // NW
CH = 25088
NCH = MP // CH
L = 16


BB = 256
NBB = B // BB


def _x2_body(x_ref, x2_ref):
    x = x_ref[...]
    x2_ref[...] = jnp.sum(x * x, axis=1, keepdims=True)


def _tc_body(xb_ref, x2_ref, mem_ref, d_ref, s1_ref, s2_ref):
    t = pl.program_id(1)
    xb = xb_ref[...]
    x2 = x2_ref[...]
    mem = mem_ref[...]
    onesv = jnp.ones((8, FD), jnp.float32)
    m2r = lax.dot_general(
        onesv, mem * mem,
        dimension_numbers=(((1,), (1,)), ((), ())),
        preferred_element_type=jnp.float32)
    mm = lax.dot_general(
        xb, mem.astype(jnp.bfloat16),
        dimension_numbers=(((1,), (1,)), ((), ())),
        preferred_element_type=jnp.float32)
    e = (m2r[:1] - 1.0) - 2.0 * mm
    col = t * TM + lax.broadcasted_iota(jnp.int32, (1, TM), 1)
    valid = col < M
    d_ref[...] = jnp.where(valid, x2 + 1.0 + e, BIG)
    ev = jnp.where(valid, e, 0.0)
    s1 = jnp.sum(ev, axis=1, keepdims=True)
    s2 = jnp.sum(ev * ev, axis=1, keepdims=True)

    @pl.when(t == 0)
    def _():
        s1_ref[...] = s1
        s2_ref[...] = s2

    @pl.when(t > 0)
    def _():
        s1_ref[...] += s1
        s2_ref[...] += s2


def _tc_distmat(x, Mem):
    x2 = pl.pallas_call(
        _x2_body,
        in_specs=[pl.BlockSpec((B, FD), lambda: (0, 0))],
        out_specs=pl.BlockSpec((B, 1), lambda: (0, 0)),
        out_shape=jax.ShapeDtypeStruct((B, 1), jnp.float32),
    )(x)
    xb = x.astype(jnp.bfloat16)
    d, s1, s2 = pl.pallas_call(
        _tc_body,
        grid=(NBB, NT),
        in_specs=[
            pl.BlockSpec((BB, FD), lambda b, t: (b, 0)),
            pl.BlockSpec((BB, 1), lambda b, t: (b, 0)),
            pl.BlockSpec((TM, FD), lambda b, t: (t, 0)),
        ],
        out_specs=[
            pl.BlockSpec((BB, TM), lambda b, t: (b, t)),
            pl.BlockSpec((BB, 1), lambda b, t: (b, 0)),
            pl.BlockSpec((BB, 1), lambda b, t: (b, 0)),
        ],
        out_shape=[
            jax.ShapeDtypeStruct((B, MP), jnp.float32),
            jax.ShapeDtypeStruct((B, 1), jnp.float32),
            jax.ShapeDtypeStruct((B, 1), jnp.float32),
        ],
        compiler_params=pltpu.CompilerParams(
            dimension_semantics=("arbitrary", "arbitrary")),
    )(xb, x2, Mem)
    return d, s1, s2, x2


def _bc(v, dt):
    return lax.broadcast_in_dim(jnp.asarray(v, dt), (L,), ())


def _sc_body(dflat, lo_h, hi_h, iw_h, labf_h,
             oL_h, oMi_h, oMx_h,
             buf0, buf1, histc, hists,
             lo_v, hi_v, iw_v, labf_v, dlab_v,
             oL_v, oMi_v, oMx_v,
             sem0, sem1, semg):
    wid = lax.axis_index("s") * 2 + lax.axis_index("c")
    base = wid * RPW
    lane = lax.broadcasted_iota(jnp.int32, (L,), 0)
    lane0 = lane == 0
    ones = _bc(1.0, jnp.float32)
    zf = jnp.zeros((L,), jnp.float32)

    pltpu.sync_copy(lo_h.at[pl.ds(base * L, RPW * L)], lo_v)
    pltpu.sync_copy(hi_h.at[pl.ds(base * L, RPW * L)], hi_v)
    pltpu.sync_copy(iw_h.at[pl.ds(base * L, RPW * L)], iw_v)
    pltpu.sync_copy(labf_h.at[pl.ds(base, RPW)], labf_v)
    pltpu.async_copy(dflat.at[labf_v], dlab_v, semg).wait()

    def row_body(r, carry):
        lo_s = lo_v[pl.ds(r * L, L)]
        hi_s = hi_v[pl.ds(r * L, L)]
        iw_s = iw_v[pl.ds(r * L, L)]
        dlab16 = dlab_v[pl.ds((r // L) * L, L)]
        lsel = _bc(r % L, jnp.int32)
        dlab = _bc(jnp.sum(jnp.where(lane == lsel, dlab16, zf)), jnp.float32)

        def zb(g, c):
            histc[pl.ds(g * L, L)] = jnp.zeros((L,), jnp.float32)
            hists[pl.ds(g * L, L)] = jnp.zeros((L,), jnp.float32)
            return c
        lax.fori_loop(0, NB // L, zb, 0)

        rowbase = (base + r) * MP
        bufs = (buf0, buf1)
        sems = (sem0, sem1)
        cp = pltpu.async_copy(dflat.at[pl.ds(rowbase, CH)], bufs[0], sems[0])
        for c in range(NCH):
            nxt = None
            if c + 1 < NCH:
                nxt = pltpu.async_copy(
                    dflat.at[pl.ds(rowbase + (c + 1) * CH, CH)],
                    bufs[(c + 1) % 2], sems[(c + 1) % 2])
            cp.wait()
            buf = bufs[c % 2]

            def grp(j, c2, _buf=buf):
                for u in range(4):
                    v = _buf[pl.ds(j * (4 * L) + u * L, L)]
                    t = (v - lo_s) * iw_s
                    idx = jnp.clip(t.astype(jnp.int32), 0, NB - 1)
                    mask = v < hi_s
                    plsc.addupdate_scatter(hists, [idx], v, mask=mask)
                    plsc.addupdate_scatter(histc, [idx], ones, mask=mask)
                return c2
            lax.fori_loop(0, CH // (4 * L), grp, 0)
            cp = nxt

        zidx = jnp.clip(((_bc(0.0, jnp.float32) - lo_s) * iw_s)
                        .astype(jnp.int32), 0, NB - 1)
        zin = (_bc(0.0, jnp.float32) < hi_s) & lane0
        plsc.addupdate_scatter(histc, [zidx], ones, mask=zin)
        lidx = jnp.clip(((dlab - lo_s) * iw_s).astype(jnp.int32), 0, NB - 1)
        lin = (dlab < hi_s) & lane0
        plsc.addupdate_scatter(hists, [lidx], -dlab, mask=lin)
        plsc.addupdate_scatter(histc, [lidx], -ones, mask=lin)

        need = jnp.float32(K)

        def scan_g(g, st):
            run_c, run_s, found, res = st
            c16 = histc[pl.ds(g * L, L)]
            s16 = hists[pl.ds(g * L, L)]
            ctot = jnp.sum(c16)
            stot = jnp.sum(s16)
            hit = jnp.logical_and(run_c + ctot >= need, found < 0.5)

            def on_hit(_):
                cc = plsc.cumsum(c16)
                crossed = (cc + _bc(run_c, jnp.float32)) >= _bc(need,
                                                                jnp.float32)
                lvec = plsc.all_reduce_ffs(crossed)
                pre = lane < lvec
                eq = lane == lvec
                cnt_b = _bc(run_c + jnp.sum(jnp.where(pre, c16, zf)),
                            jnp.float32)
                sum_b = _bc(run_s + jnp.sum(jnp.where(pre, s16, zf)),
                            jnp.float32)
                selc = _bc(jnp.sum(jnp.where(eq, c16, zf)), jnp.float32)
                sels = _bc(jnp.sum(jnp.where(eq, s16, zf)), jnp.float32)
                take = jnp.clip(_bc(need, jnp.float32) - cnt_b, zf, selc)
                return sum_b + take * sels / jnp.maximum(selc, ones)

            res = lax.cond(hit, on_hit, lambda _: res, 0)
            found = jnp.where(hit, jnp.float32(1.0), found)
            return (run_c + ctot, run_s + stot, found, res)

        run_c, run_s, found, res = lax.fori_loop(
            0, NB // L, scan_g,
            (jnp.float32(0.0), jnp.float32(0.0), jnp.float32(0.0), zf))
        sel_sum = jnp.where(found > 0.5, res, _bc(run_s, jnp.float32))

        maxl = sel_sum / _bc(float(K - 1), jnp.float32)
        minl = (jnp.clip(dlab, 1e-12, 1e12)
                + _bc(float((M - 1) * 1e-12), jnp.float32))
        lossr = jnp.maximum(minl - maxl + _bc(MARGIN, jnp.float32), zf)
        oL_v[pl.ds(r * L, L)] = lossr
        oMi_v[pl.ds(r * L, L)] = minl
        oMx_v[pl.ds(r * L, L)] = maxl
        return carry

    lax.fori_loop(0, RPW, row_body, 0)

    pltpu.sync_copy(oL_v, oL_h.at[pl.ds(base * L, RPW * L)])
    pltpu.sync_copy(oMi_v, oMi_h.at[pl.ds(base * L, RPW * L)])
    pltpu.sync_copy(oMx_v, oMx_h.at[pl.ds(base * L, RPW * L)])


def _sc_select(dflat, lo, hi, iw, labf):
    mesh = plsc.VectorSubcoreMesh(core_axis_name="c", subcore_axis_name="s")
    f32 = jnp.float32
    fn = pl.kernel(
        _sc_body,
        out_type=(jax.ShapeDtypeStruct((B * L,), f32),
                  jax.ShapeDtypeStruct((B * L,), f32),
                  jax.ShapeDtypeStruct((B * L,), f32)),
        mesh=mesh,
        compiler_params=pltpu.CompilerParams(needs_layout_passes=False),
        scratch_types=[
            pltpu.VMEM((CH,), f32),
            pltpu.VMEM((CH,), f32),
            pltpu.VMEM((NB,), f32),
            pltpu.VMEM((NB,), f32),
            pltpu.VMEM((RPW * L,), f32),
            pltpu.VMEM((RPW * L,), f32),
            pltpu.VMEM((RPW * L,), f32),
            pltpu.VMEM((RPW,), jnp.int32),
            pltpu.VMEM((RPW,), f32),
            pltpu.VMEM((RPW * L,), f32),
            pltpu.VMEM((RPW * L,), f32),
            pltpu.VMEM((RPW * L,), f32),
            pltpu.SemaphoreType.DMA,
            pltpu.SemaphoreType.DMA,
            pltpu.SemaphoreType.DMA,
        ],
    )
    return fn(dflat, lo, hi, iw, labf)


def kernel(x, labels, Mem):
    d, s1, s2, x2 = _tc_distmat(x, Mem)
    s1 = s1[:, 0]
    s2 = s2[:, 0]
    x2 = x2[:, 0]
    mu_e = s1 / M
    sig = jnp.sqrt(jnp.maximum(s2 / M - mu_e * mu_e, 1e-12))
    center = x2 + 1.0 + mu_e
    lo = center - Z_LO * sig
    hi = center - Z_HI * sig
    iw = NB / (hi - lo)

    def splat(v):
        return jnp.broadcast_to(v[:, None], (B, L)).reshape(B * L)

    labf = jnp.arange(B, dtype=jnp.int32) * MP + labels.astype(jnp.int32)
    dflat = d.reshape((B * MP,))
    loss_v, minl_v, maxl_v = _sc_select(
        dflat, splat(lo), splat(hi), splat(iw), labf)
    return (jnp.mean(loss_v.reshape(B, L)[:, 0]),
            jnp.mean(minl_v.reshape(B, L)[:, 0]),
            jnp.mean(maxl_v.reshape(B, L)[:, 0]))


# speedup vs baseline: 17.1602x; 1.0002x over previous
import functools

import jax
import jax.numpy as jnp
from jax import lax
from jax.experimental import pallas as pl
from jax.experimental.pallas import tpu as pltpu
from jax.experimental.pallas import tpu_sc as plsc

B = 1024
M = 100000
FD = 512
K = 1000
MARGIN = 1.0

TM = 2048
NT = 49
MP = NT * TM

NB = 1024
Z_LO = 3.5
Z_HI = 1.25
BIG = 1.0e9

NW = 32
RPW = B // NW
CH = 25088
NCH = MP // CH
L = 16


BB = 256
NBB = B // BB


def _x2_body(x_ref, x2_ref):
    x = x_ref[...]
    x2_ref[...] = jnp.sum(x * x, axis=1, keepdims=True)


def _tc_body(xb_ref, x2_ref, mem_ref, d_ref, s1_ref, s2_ref):
    t = pl.program_id(1)
    xb = xb_ref[...]
    x2 = x2_ref[...]
    mem = mem_ref[...]
    onesv = jnp.ones((8, FD), jnp.float32)
    m2r = lax.dot_general(
        onesv, mem * mem,
        dimension_numbers=(((1,), (1,)), ((), ())),
        preferred_element_type=jnp.float32)
    mm = lax.dot_general(
        xb, mem.astype(jnp.bfloat16),
        dimension_numbers=(((1,), (1,)), ((), ())),
        preferred_element_type=jnp.float32)
    e = (m2r[:1] - 1.0) - 2.0 * mm
    col = t * TM + lax.broadcasted_iota(jnp.int32, (1, TM), 1)
    valid = col < M
    d_ref[...] = jnp.where(valid, x2 + 1.0 + e, BIG)
    ev = jnp.where(valid, e, 0.0)
    s1 = jnp.sum(ev, axis=1, keepdims=True)
    s2 = jnp.sum(ev * ev, axis=1, keepdims=True)

    @pl.when(t == 0)
    def _():
        s1_ref[...] = s1
        s2_ref[...] = s2

    @pl.when(t > 0)
    def _():
        s1_ref[...] += s1
        s2_ref[...] += s2


def _tc_distmat(x, Mem):
    x2 = pl.pallas_call(
        _x2_body,
        in_specs=[pl.BlockSpec((B, FD), lambda: (0, 0))],
        out_specs=pl.BlockSpec((B, 1), lambda: (0, 0)),
        out_shape=jax.ShapeDtypeStruct((B, 1), jnp.float32),
    )(x)
    xb = x.astype(jnp.bfloat16)
    d, s1, s2 = pl.pallas_call(
        _tc_body,
        grid=(NBB, NT),
        in_specs=[
            pl.BlockSpec((BB, FD), lambda b, t: (b, 0)),
            pl.BlockSpec((BB, 1), lambda b, t: (b, 0)),
            pl.BlockSpec((TM, FD), lambda b, t: (t, 0)),
        ],
        out_specs=[
            pl.BlockSpec((BB, TM), lambda b, t: (b, t)),
            pl.BlockSpec((BB, 1), lambda b, t: (b, 0)),
            pl.BlockSpec((BB, 1), lambda b, t: (b, 0)),
        ],
        out_shape=[
            jax.ShapeDtypeStruct((B, MP), jnp.float32),
            jax.ShapeDtypeStruct((B, 1), jnp.float32),
            jax.ShapeDtypeStruct((B, 1), jnp.float32),
        ],
        compiler_params=pltpu.CompilerParams(
            dimension_semantics=("arbitrary", "arbitrary")),
    )(xb, x2, Mem)
    return d, s1, s2, x2


def _bc(v, dt):
    return lax.broadcast_in_dim(jnp.asarray(v, dt), (L,), ())


def _sc_body(d_h, lo_h, hi_h, iw_h, labf_h,
             oL_h, oMi_h, oMx_h,
             buf0, buf1, histc, hists,
             lo_v, hi_v, iw_v, labf_v, dlab_v,
             oL_v, oMi_v, oMx_v,
             sem0, sem1, semg):
    wid = lax.axis_index("s") * 2 + lax.axis_index("c")
    base = wid * RPW
    lane = lax.broadcasted_iota(jnp.int32, (L,), 0)
    lane0 = lane == 0
    ones = _bc(1.0, jnp.float32)
    zf = jnp.zeros((L,), jnp.float32)

    pltpu.sync_copy(lo_h.at[pl.ds(base * L, RPW * L)], lo_v)
    pltpu.sync_copy(hi_h.at[pl.ds(base * L, RPW * L)], hi_v)
    pltpu.sync_copy(iw_h.at[pl.ds(base * L, RPW * L)], iw_v)
    pltpu.sync_copy(labf_h.at[pl.ds(base, RPW)], labf_v)

    def row_body(r, carry):
        lo_s = lo_v[pl.ds(r * L, L)]
        hi_s = hi_v[pl.ds(r * L, L)]
        iw_s = iw_v[pl.ds(r * L, L)]
        lab16 = labf_v[pl.ds((r // L) * L, L)]
        lsel = _bc(r % L, jnp.int32)
        labc = jnp.sum(jnp.where(lane == lsel, lab16, jnp.zeros((L,),
                                                                jnp.int32)))
        laba = (labc // 8) * 8
        cpl = pltpu.async_copy(d_h.at[base + r, pl.ds(laba, L)],
                               dlab_v, semg)
        loiw = lo_s * iw_s

        def zb(g, c):
            histc[pl.ds(g * L, L)] = jnp.zeros((L,), jnp.float32)
            hists[pl.ds(g * L, L)] = jnp.zeros((L,), jnp.float32)
            return c
        lax.fori_loop(0, NB // L, zb, 0)

        row = base + r
        bufs = (buf0, buf1)
        sems = (sem0, sem1)
        cp = pltpu.async_copy(d_h.at[row, pl.ds(0, CH)], bufs[0], sems[0])
        for c in range(NCH):
            nxt = None
            if c + 1 < NCH:
                nxt = pltpu.async_copy(
                    d_h.at[row, pl.ds((c + 1) * CH, CH)],
                    bufs[(c + 1) % 2], sems[(c + 1) % 2])
            cp.wait()
            buf = bufs[c % 2]

            def grp(j, c2, _buf=buf):
                for u in range(4):
                    v = _buf[pl.ds(j * (4 * L) + u * L, L)]
                    t = v * iw_s - loiw
                    idx = jnp.clip(t.astype(jnp.int32), 0, NB - 1)
                    mask = v < hi_s
                    plsc.addupdate_scatter(hists, [idx], v, mask=mask)
                    plsc.addupdate_scatter(histc, [idx], ones, mask=mask)
                return c2
            lax.fori_loop(0, CH // (4 * L), grp, 0)
            cp = nxt

        cpl.wait()
        dlab16 = dlab_v[pl.ds(0, L)]
        dlab = _bc(jnp.sum(jnp.where(lane == _bc(labc - laba, jnp.int32),
                                     dlab16, zf)), jnp.float32)
        zidx = jnp.clip((_bc(0.0, jnp.float32) * iw_s - loiw)
                        .astype(jnp.int32), 0, NB - 1)
        zin = (_bc(0.0, jnp.float32) < hi_s) & lane0
        plsc.addupdate_scatter(histc, [zidx], ones, mask=zin)
        lidx = jnp.clip((dlab * iw_s - loiw).astype(jnp.int32), 0, NB - 1)
        lin = (dlab < hi_s) & lane0
        plsc.addupdate_scatter(hists, [lidx], -dlab, mask=lin)
        plsc.addupdate_scatter(histc, [lidx], -ones, mask=lin)

        need = jnp.float32(K)

        def scan_g(g, st):
            run_c, run_s, found, res = st
            c16 = histc[pl.ds(g * L, L)]
            s16 = hists[pl.ds(g * L, L)]
            ctot = jnp.sum(c16)
            stot = jnp.sum(s16)
            hit = jnp.logical_and(run_c + ctot >= need, found < 0.5)

            def on_hit(_):
                cc = plsc.cumsum(c16)
                crossed = (cc + _bc(run_c, jnp.float32)) >= _bc(need,
                                                                jnp.float32)
                lvec = plsc.all_reduce_ffs(crossed)
                pre = lane < lvec
                eq = lane == lvec
                cnt_b = _bc(run_c + jnp.sum(jnp.where(pre, c16, zf)),
                            jnp.float32)
                sum_b = _bc(run_s + jnp.sum(jnp.where(pre, s16, zf)),
                            jnp.float32)
                selc = _bc(jnp.sum(jnp.where(eq, c16, zf)), jnp.float32)
                sels = _bc(jnp.sum(jnp.where(eq, s16, zf)), jnp.float32)
                take = jnp.clip(_bc(need, jnp.float32) - cnt_b, zf, selc)
                return sum_b + take * sels / jnp.maximum(selc, ones)

            res = lax.cond(hit, on_hit, lambda _: res, 0)
            found = jnp.where(hit, jnp.float32(1.0), found)
            return (run_c + ctot, run_s + stot, found, res)

        run_c, run_s, found, res = lax.fori_loop(
            0, NB // L, scan_g,
            (jnp.float32(0.0), jnp.float32(0.0), jnp.float32(0.0), zf))
        sel_sum = jnp.where(found > 0.5, res, _bc(run_s, jnp.float32))

        maxl = sel_sum / _bc(float(K - 1), jnp.float32)
        minl = (jnp.clip(dlab, 1e-12, 1e12)
                + _bc(float((M - 1) * 1e-12), jnp.float32))
        lossr = jnp.maximum(minl - maxl + _bc(MARGIN, jnp.float32), zf)
        oL_v[pl.ds(r * L, L)] = lossr
        oMi_v[pl.ds(r * L, L)] = minl
        oMx_v[pl.ds(r * L, L)] = maxl
        return carry

    lax.fori_loop(0, RPW, row_body, 0)

    pltpu.sync_copy(oL_v, oL_h.at[pl.ds(base * L, RPW * L)])
    pltpu.sync_copy(oMi_v, oMi_h.at[pl.ds(base * L, RPW * L)])
    pltpu.sync_copy(oMx_v, oMx_h.at[pl.ds(base * L, RPW * L)])


def _sc_select(d, lo, hi, iw, labf):
    mesh = plsc.VectorSubcoreMesh(core_axis_name="c", subcore_axis_name="s")
    f32 = jnp.float32
    fn = pl.kernel(
        _sc_body,
        out_type=(jax.ShapeDtypeStruct((B * L,), f32),
                  jax.ShapeDtypeStruct((B * L,), f32),
                  jax.ShapeDtypeStruct((B * L,), f32)),
        mesh=mesh,
        compiler_params=pltpu.CompilerParams(needs_layout_passes=False,
                                             use_tc_tiling_on_sc=False),
        scratch_types=[
            pltpu.VMEM((CH,), f32),
            pltpu.VMEM((CH,), f32),
            pltpu.VMEM((NB,), f32),
            pltpu.VMEM((NB,), f32),
            pltpu.VMEM((RPW * L,), f32),
            pltpu.VMEM((RPW * L,), f32),
            pltpu.VMEM((RPW * L,), f32),
            pltpu.VMEM((RPW,), jnp.int32),
            pltpu.VMEM((L,), f32),
            pltpu.VMEM((RPW * L,), f32),
            pltpu.VMEM((RPW * L,), f32),
            pltpu.VMEM((RPW * L,), f32),
            pltpu.SemaphoreType.DMA,
            pltpu.SemaphoreType.DMA,
            pltpu.SemaphoreType.DMA,
        ],
    )
    return fn(d, lo, hi, iw, labf)


def kernel(x, labels, Mem):
    d, s1, s2, x2 = _tc_distmat(x, Mem)
    s1 = s1[:, 0]
    s2 = s2[:, 0]
    x2 = x2[:, 0]
    mu_e = s1 / M
    sig = jnp.sqrt(jnp.maximum(s2 / M - mu_e * mu_e, 1e-12))
    center = x2 + 1.0 + mu_e
    lo = center - Z_LO * sig
    hi = center - Z_HI * sig
    iw = NB / (hi - lo)

    def splat(v):
        return jnp.broadcast_to(v[:, None], (B, L)).reshape(B * L)

    labf = labels.astype(jnp.int32)
    loss_v, minl_v, maxl_v = _sc_select(
        d, splat(lo), splat(hi), splat(iw), labf)
    return (jnp.mean(loss_v.reshape(B, L)[:, 0]),
            jnp.mean(minl_v.reshape(B, L)[:, 0]),
            jnp.mean(maxl_v.reshape(B, L)[:, 0]))


# speedup vs baseline: 44.1906x; 2.5752x over previous
import functools

import jax
import jax.numpy as jnp
from jax import lax
from jax.experimental import pallas as pl
from jax.experimental.pallas import tpu as pltpu
from jax.experimental.pallas import tpu_sc as plsc

B = 1024
M = 100000
FD = 512
K = 1000
MARGIN = 1.0

TM = 2048
NT = 49
MP = NT * TM

NB = 1024
Z_LO = 3.5
Z_HI = 1.25
BIG = 1.0e9

NW = 32
RPW = B // NW
CH = 25088
NCH = MP // CH
L = 16


BB = 256
NBB = B // BB


def _x2_body(x_ref, x2_ref):
    x = x_ref[...]
    x2_ref[...] = jnp.sum(x * x, axis=1, keepdims=True)


def _tc_body(xb_ref, x2_ref, mem_ref, d_ref, s1_ref, s2_ref):
    t = pl.program_id(1)
    xb = xb_ref[...]
    x2 = x2_ref[...]
    mem = mem_ref[...]
    onesv = jnp.ones((8, FD), jnp.float32)
    m2r = lax.dot_general(
        onesv, mem * mem,
        dimension_numbers=(((1,), (1,)), ((), ())),
        preferred_element_type=jnp.float32)
    mm = lax.dot_general(
        xb, mem.astype(jnp.bfloat16),
        dimension_numbers=(((1,), (1,)), ((), ())),
        preferred_element_type=jnp.float32)
    e = (m2r[:1] - 1.0) - 2.0 * mm
    col = t * TM + lax.broadcasted_iota(jnp.int32, (1, TM), 1)
    valid = col < M
    d_ref[...] = jnp.where(valid, x2 + 1.0 + e, BIG)
    ev = jnp.where(valid, e, 0.0)
    s1 = jnp.sum(ev, axis=1, keepdims=True)
    s2 = jnp.sum(ev * ev, axis=1, keepdims=True)

    @pl.when(t == 0)
    def _():
        s1_ref[...] = s1
        s2_ref[...] = s2

    @pl.when(t > 0)
    def _():
        s1_ref[...] += s1
        s2_ref[...] += s2


def _tc_distmat(x, Mem):
    x2 = pl.pallas_call(
        _x2_body,
        in_specs=[pl.BlockSpec((B, FD), lambda: (0, 0))],
        out_specs=pl.BlockSpec((B, 1), lambda: (0, 0)),
        out_shape=jax.ShapeDtypeStruct((B, 1), jnp.float32),
    )(x)
    xb = x.astype(jnp.bfloat16)
    d, s1, s2 = pl.pallas_call(
        _tc_body,
        grid=(NBB, NT),
        in_specs=[
            pl.BlockSpec((BB, FD), lambda b, t: (b, 0)),
            pl.BlockSpec((BB, 1), lambda b, t: (b, 0)),
            pl.BlockSpec((TM, FD), lambda b, t: (t, 0)),
        ],
        out_specs=[
            pl.BlockSpec((BB, TM), lambda b, t: (b, t)),
            pl.BlockSpec((BB, 1), lambda b, t: (b, 0)),
            pl.BlockSpec((BB, 1), lambda b, t: (b, 0)),
        ],
        out_shape=[
            jax.ShapeDtypeStruct((B, MP), jnp.float32),
            jax.ShapeDtypeStruct((B, 1), jnp.float32),
            jax.ShapeDtypeStruct((B, 1), jnp.float32),
        ],
        compiler_params=pltpu.CompilerParams(
            dimension_semantics=("arbitrary", "arbitrary")),
    )(xb, x2, Mem)
    return d, s1, s2, x2


def _bc(v, dt):
    return lax.broadcast_in_dim(jnp.asarray(v, dt), (L,), ())


def _sc_body(d_h, lo_h, hi_h, iw_h, labf_h,
             oL_h, oMi_h, oMx_h,
             buf0, buf1, histc, hists,
             lo_v, hi_v, iw_v, labf_v, dlab_v,
             oL_v, oMi_v, oMx_v,
             sem0, sem1, semg):
    wid = lax.axis_index("s") * 2 + lax.axis_index("c")
    base = wid * RPW
    lane = lax.broadcasted_iota(jnp.int32, (L,), 0)
    lane0 = lane == 0
    ones = _bc(1.0, jnp.float32)
    zf = jnp.zeros((L,), jnp.float32)

    pltpu.sync_copy(lo_h.at[pl.ds(base * L, RPW * L)], lo_v)
    pltpu.sync_copy(hi_h.at[pl.ds(base * L, RPW * L)], hi_v)
    pltpu.sync_copy(iw_h.at[pl.ds(base * L, RPW * L)], iw_v)
    pltpu.sync_copy(labf_h.at[pl.ds(base, RPW)], labf_v)

    def row_body(r, carry):
        lo_s = lo_v[pl.ds(r * L, L)]
        hi_s = hi_v[pl.ds(r * L, L)]
        iw_s = iw_v[pl.ds(r * L, L)]
        lab16 = labf_v[pl.ds((r // L) * L, L)]
        lsel = _bc(r % L, jnp.int32)
        labc = jnp.sum(jnp.where(lane == lsel, lab16, jnp.zeros((L,),
                                                                jnp.int32)))
        laba = (labc // 8) * 8
        cpl = pltpu.async_copy(d_h.at[base + r, pl.ds(laba, L)],
                               dlab_v, semg)
        loiw = lo_s * iw_s

        def zb(g, c):
            histc[pl.ds(g * L, L)] = jnp.zeros((L,), jnp.float32)
            hists[pl.ds(g * L, L)] = jnp.zeros((L,), jnp.float32)
            return c
        lax.fori_loop(0, NB // L, zb, 0)

        row = base + r
        bufs = (buf0, buf1)
        sems = (sem0, sem1)
        cp = pltpu.async_copy(d_h.at[row, pl.ds(0, CH)], bufs[0], sems[0])
        for c in range(NCH):
            nxt = None
            if c + 1 < NCH:
                nxt = pltpu.async_copy(
                    d_h.at[row, pl.ds((c + 1) * CH, CH)],
                    bufs[(c + 1) % 2], sems[(c + 1) % 2])
            cp.wait()
            buf = bufs[c % 2]

            def grp(j, _buf=buf):
                v = _buf[pl.ds(j * L, L)]
                t = v * iw_s - loiw
                idx = jnp.clip(t.astype(jnp.int32), 0, NB - 1)
                mask = v < hi_s
                plsc.addupdate_scatter(hists, [idx], v, mask=mask)
                plsc.addupdate_scatter(histc, [idx], ones, mask=mask)
            plsc.parallel_loop(0, CH // L, 1, unroll=8)(grp)
            cp = nxt

        cpl.wait()
        dlab16 = dlab_v[pl.ds(0, L)]
        dlab = _bc(jnp.sum(jnp.where(lane == _bc(labc - laba, jnp.int32),
                                     dlab16, zf)), jnp.float32)
        zidx = jnp.clip((_bc(0.0, jnp.float32) * iw_s - loiw)
                        .astype(jnp.int32), 0, NB - 1)
        zin = (_bc(0.0, jnp.float32) < hi_s) & lane0
        plsc.addupdate_scatter(histc, [zidx], ones, mask=zin)
        lidx = jnp.clip((dlab * iw_s - loiw).astype(jnp.int32), 0, NB - 1)
        lin = (dlab < hi_s) & lane0
        plsc.addupdate_scatter(hists, [lidx], -dlab, mask=lin)
        plsc.addupdate_scatter(histc, [lidx], -ones, mask=lin)

        need = jnp.float32(K)

        def scan_g(g, st):
            run_c, run_s, found, res = st
            c16 = histc[pl.ds(g * L, L)]
            s16 = hists[pl.ds(g * L, L)]
            ctot = jnp.sum(c16)
            stot = jnp.sum(s16)
            hit = jnp.logical_and(run_c + ctot >= need, found < 0.5)

            def on_hit(_):
                cc = plsc.cumsum(c16)
                crossed = (cc + _bc(run_c, jnp.float32)) >= _bc(need,
                                                                jnp.float32)
                lvec = plsc.all_reduce_ffs(crossed)
                pre = lane < lvec
                eq = lane == lvec
                cnt_b = _bc(run_c + jnp.sum(jnp.where(pre, c16, zf)),
                            jnp.float32)
                sum_b = _bc(run_s + jnp.sum(jnp.where(pre, s16, zf)),
                            jnp.float32)
                selc = _bc(jnp.sum(jnp.where(eq, c16, zf)), jnp.float32)
                sels = _bc(jnp.sum(jnp.where(eq, s16, zf)), jnp.float32)
                take = jnp.clip(_bc(need, jnp.float32) - cnt_b, zf, selc)
                return sum_b + take * sels / jnp.maximum(selc, ones)

            res = lax.cond(hit, on_hit, lambda _: res, 0)
            found = jnp.where(hit, jnp.float32(1.0), found)
            return (run_c + ctot, run_s + stot, found, res)

        run_c, run_s, found, res = lax.fori_loop(
            0, NB // L, scan_g,
            (jnp.float32(0.0), jnp.float32(0.0), jnp.float32(0.0), zf))
        sel_sum = jnp.where(found > 0.5, res, _bc(run_s, jnp.float32))

        maxl = sel_sum / _bc(float(K - 1), jnp.float32)
        minl = (jnp.clip(dlab, 1e-12, 1e12)
                + _bc(float((M - 1) * 1e-12), jnp.float32))
        lossr = jnp.maximum(minl - maxl + _bc(MARGIN, jnp.float32), zf)
        oL_v[pl.ds(r * L, L)] = lossr
        oMi_v[pl.ds(r * L, L)] = minl
        oMx_v[pl.ds(r * L, L)] = maxl
        return carry

    lax.fori_loop(0, RPW, row_body, 0)

    pltpu.sync_copy(oL_v, oL_h.at[pl.ds(base * L, RPW * L)])
    pltpu.sync_copy(oMi_v, oMi_h.at[pl.ds(base * L, RPW * L)])
    pltpu.sync_copy(oMx_v, oMx_h.at[pl.ds(base * L, RPW * L)])


def _sc_select(d, lo, hi, iw, labf):
    mesh = plsc.VectorSubcoreMesh(core_axis_name="c", subcore_axis_name="s")
    f32 = jnp.float32
    fn = pl.kernel(
        _sc_body,
        out_type=(jax.ShapeDtypeStruct((B * L,), f32),
                  jax.ShapeDtypeStruct((B * L,), f32),
                  jax.ShapeDtypeStruct((B * L,), f32)),
        mesh=mesh,
        compiler_params=pltpu.CompilerParams(needs_layout_passes=False,
                                             use_tc_tiling_on_sc=False),
        scratch_types=[
            pltpu.VMEM((CH,), f32),
            pltpu.VMEM((CH,), f32),
            pltpu.VMEM((NB,), f32),
            pltpu.VMEM((NB,), f32),
            pltpu.VMEM((RPW * L,), f32),
            pltpu.VMEM((RPW * L,), f32),
            pltpu.VMEM((RPW * L,), f32),
            pltpu.VMEM((RPW,), jnp.int32),
            pltpu.VMEM((L,), f32),
            pltpu.VMEM((RPW * L,), f32),
            pltpu.VMEM((RPW * L,), f32),
            pltpu.VMEM((RPW * L,), f32),
            pltpu.SemaphoreType.DMA,
            pltpu.SemaphoreType.DMA,
            pltpu.SemaphoreType.DMA,
        ],
    )
    return fn(d, lo, hi, iw, labf)


def kernel(x, labels, Mem):
    d, s1, s2, x2 = _tc_distmat(x, Mem)
    s1 = s1[:, 0]
    s2 = s2[:, 0]
    x2 = x2[:, 0]
    mu_e = s1 / M
    sig = jnp.sqrt(jnp.maximum(s2 / M - mu_e * mu_e, 1e-12))
    center = x2 + 1.0 + mu_e
    lo = center - Z_LO * sig
    hi = center - Z_HI * sig
    iw = NB / (hi - lo)

    def splat(v):
        return jnp.broadcast_to(v[:, None], (B, L)).reshape(B * L)

    labf = labels.astype(jnp.int32)
    loss_v, minl_v, maxl_v = _sc_select(
        d, splat(lo), splat(hi), splat(iw), labf)
    return (jnp.mean(loss_v.reshape(B, L)[:, 0]),
            jnp.mean(minl_v.reshape(B, L)[:, 0]),
            jnp.mean(maxl_v.reshape(B, L)[:, 0]))
